# baseline (speedup 1.0000x reference)
import functools

import jax
import jax.numpy as jnp
from jax import lax
from jax.experimental import pallas as pl
from jax.experimental.pallas import tpu as pltpu
from jax.experimental.pallas import tpu_sc as plsc

_NE = 16
_MIN_EXPERTS = 2
_L = 16
_NW = 32


def _logits_body(x_ref, w_ref, g_ref, lg_ref):
    x = x_ref[...]
    w = w_ref[...]
    g = g_ref[...]

    wn = w * jax.lax.rsqrt(jnp.maximum(jnp.sum(w * w, axis=0, keepdims=True),
                                       1e-24))
    rn = jnp.sqrt(jnp.sum(x * x, axis=1, keepdims=True))
    xn = x / jnp.maximum(rn, 1e-12)
    xw = jnp.dot(xn.astype(jnp.bfloat16), wn.astype(jnp.bfloat16),
                 preferred_element_type=jnp.float32)
    lg_ref[...] = xw - jax.nn.sigmoid(g)


def _tc_logits(flat, sim_matrix, g2, tile_m):
    n, c = flat.shape
    return pl.pallas_call(
        _logits_body,
        grid=(n // tile_m,),
        in_specs=[
            pl.BlockSpec((tile_m, c), lambda i: (i, 0)),
            pl.BlockSpec((c, _NE), lambda i: (0, 0)),
            pl.BlockSpec((1, _NE), lambda i: (0, 0)),
        ],
        out_specs=pl.BlockSpec((tile_m, _NE), lambda i: (i, 0)),
        out_shape=jax.ShapeDtypeStruct((n, _NE), jnp.float32),
        compiler_params=pltpu.CompilerParams(
            dimension_semantics=("parallel",),
        ),
    )(flat, sim_matrix, g2)


def _routing_groups(lv, rw_v, am_v, n_groups):
    iota = lax.iota(jnp.int32, _L)
    neg_inf = jnp.full((_L,), -jnp.inf, jnp.float32)
    zero_f = jnp.zeros((_L,), jnp.float32)

    def group(g, carry):
        rowbase = (g * _L + iota) * _NE
        cols = [jnp.full((_L,), e, jnp.int32) for e in range(_NE)]
        xs = [plsc.load_gather(lv, [rowbase + cols[e]]) for e in range(_NE)]

        act = [x > zero_f for x in xs]
        nact = jnp.zeros((_L,), jnp.float32)
        for e in range(_NE):
            nact = nact + jnp.where(act[e], 1.0, 0.0)
        inactive = nact == zero_f

        m1 = neg_inf
        m2 = neg_inf
        i1 = jnp.full((_L,), 0, jnp.int32)
        i2 = jnp.full((_L,), 0, jnp.int32)
        for e in range(_NE):
            x = xs[e]
            gt1 = x > m1
            gt2 = x > m2
            e_v = jnp.full((_L,), e, jnp.int32)
            i2 = jnp.where(gt1, i1, jnp.where(gt2, e_v, i2))
            m2 = jnp.where(gt1, m1, jnp.where(gt2, x, m2))
            i1 = jnp.where(gt1, e_v, i1)
            m1 = jnp.where(gt1, x, m1)

        masks = []
        ml = []
        e_ids = [jnp.full((_L,), e, jnp.int32) for e in range(_NE)]
        for e in range(_NE):
            fb = (i1 == e_ids[e]) | (i2 == e_ids[e])
            me = jnp.where(inactive, fb, act[e])
            masks.append(me)
            gated = jnp.maximum(xs[e], zero_f)
            ml.append(jnp.where(me, gated, neg_inf))
        mx = ml[0]
        for e in range(1, _NE):
            mx = jnp.maximum(mx, ml[e])
        ps = [jnp.exp(v - mx) for v in ml]
        s = ps[0]
        for e in range(1, _NE):
            s = s + ps[e]
        inv = 1.0 / s
        for e in range(_NE):
            plsc.store_scatter(rw_v, [rowbase + cols[e]], ps[e] * inv)
            plsc.store_scatter(am_v, [rowbase + cols[e]],
                               jnp.where(masks[e], 1.0, 0.0))
        return carry

    lax.fori_loop(0, n_groups, group, 0)


def _sc_routing(logits_flat, n):
    tpw = n // _NW
    fpw = tpw * _NE
    mesh = plsc.VectorSubcoreMesh(core_axis_name="c", subcore_axis_name="s")

    @functools.partial(
        pl.kernel,
        out_type=[
            jax.ShapeDtypeStruct((n * _NE,), jnp.float32),
            jax.ShapeDtypeStruct((n * _NE,), jnp.float32),
        ],
        mesh=mesh,
        scratch_types=[
            pltpu.VMEM((fpw,), jnp.float32),
            pltpu.VMEM((fpw,), jnp.float32),
            pltpu.VMEM((fpw,), jnp.float32),
        ],
        compiler_params=pltpu.CompilerParams(needs_layout_passes=False),
    )
    def routing(lg_hbm, rw_hbm, am_hbm, lv, rw_v, am_v):
        wid = lax.axis_index("s") * 2 + lax.axis_index("c")
        base = wid * fpw
        pltpu.sync_copy(lg_hbm.at[pl.ds(base, fpw)], lv)
        _routing_groups(lv, rw_v, am_v, tpw // _L)
        pltpu.sync_copy(rw_v, rw_hbm.at[pl.ds(base, fpw)])
        pltpu.sync_copy(am_v, am_hbm.at[pl.ds(base, fpw)])

    return routing(logits_flat)


@jax.jit
def kernel(hidden_states, sim_matrix, gates):
    b, t, c = hidden_states.shape
    n = b * t
    flat = hidden_states.reshape(n, c)
    g2 = gates.reshape(1, _NE)

    logits = _tc_logits(flat, sim_matrix, g2, tile_m=2048)
    rw, am = _sc_routing(logits.reshape(-1), n)
    return rw.reshape(n, _NE), logits, am.reshape(n, _NE)
